# Initial kernel scaffold; baseline (speedup 1.0000x reference)
import jax
import jax.numpy as jnp
from jax.experimental import pallas as pl


def kernel(x, batch, category, W1, b1, g1, be1, Wml, bml, gml, beml, Wmg, bmg, gmg, bemg, Wsl, bsl, gsl, besl, Wsg, bsg, gsg, besg, W3, b3, W4, b4):
    raise NotImplementedError("write your pallas kernel here")


# speedup vs baseline: 1.0152x; 1.0152x over previous
import functools

import jax
import jax.numpy as jnp
from jax.experimental import pallas as pl
from jax.experimental.pallas import tpu as pltpu

EPS = 1e-5
SLOPE = 0.01
NB = 8
P = 2048
NPTS = NB * P
KNN = 16


def _bn(h, g, be):
    m = jnp.mean(h, axis=0, keepdims=True)
    v = jnp.mean((h - m) ** 2, axis=0, keepdims=True)
    return (h - m) * jax.lax.rsqrt(v + EPS) * g + be


def _mlp(h, W, b, g, be):
    h = jnp.dot(h, W, preferred_element_type=jnp.float32) + b
    h = jnp.where(h > 0.0, h, SLOPE * h)
    return _bn(h, g, be)


def _front_body(x_ref, W1_ref, b1_ref, g1_ref, be1_ref,
                Wml_ref, bml_ref, gml_ref, beml_ref,
                Wsl_ref, bsl_ref, gsl_ref, besl_ref,
                mu_ref, sig_ref):
    x = x_ref[...]
    h = _mlp(x, W1_ref[...], b1_ref[...], g1_ref[...], be1_ref[...])
    mu_ref[...] = _mlp(h, Wml_ref[...], bml_ref[...], gml_ref[...], beml_ref[...])
    sig_ref[...] = _mlp(h, Wsl_ref[...], bsl_ref[...], gsl_ref[...], besl_ref[...])


def _tail_body(aggm_ref, aggs_ref,
               Wmg_ref, bmg_ref, gmg_ref, bemg_ref,
               Wsg_ref, bsg_ref, gsg_ref, besg_ref,
               W3_ref, b3_ref, W4_ref, b4_ref,
               out_ref, zmu_ref, zsig_ref):
    zm = _mlp(aggm_ref[...], Wmg_ref[...], bmg_ref[...], gmg_ref[...], bemg_ref[...])
    zs = _mlp(aggs_ref[...], Wsg_ref[...], bsg_ref[...], gsg_ref[...], besg_ref[...])
    zm = jnp.max(zm.reshape(NB, P, 128), axis=1)
    zs = jnp.max(zs.reshape(NB, P, 128), axis=1)
    zmu_ref[...] = zm
    zsig_ref[...] = jnp.minimum(zs, 3.0)
    o = jnp.dot(zm, W3_ref[...], preferred_element_type=jnp.float32) + b3_ref[...]
    o = jnp.maximum(o, 0.0)
    out_ref[...] = jnp.dot(o, W4_ref[...], preferred_element_type=jnp.float32) + b4_ref[...]


def _knn0(x):
    xg = x.reshape(NB, P, 3)
    d2 = jnp.sum(xg * xg, axis=-1)
    dist = d2[:, :, None] + d2[:, None, :] - 2.0 * jnp.einsum('bid,bjd->bij', xg, xg)
    dist = dist + jnp.eye(P, dtype=x.dtype)[None] * 1e10
    _, idx = jax.lax.top_k(-dist, KNN)
    offset = (jnp.arange(NB, dtype=jnp.int32) * P)[:, None, None]
    return (idx.astype(jnp.int32) + offset).reshape(NPTS, KNN)


def kernel(x, batch, category, W1, b1, g1, be1,
           Wml, bml, gml, beml, Wmg, bmg, gmg, bemg,
           Wsl, bsl, gsl, besl, Wsg, bsg, gsg, besg,
           W3, b3, W4, b4):
    r = lambda v: v.reshape(1, -1)
    f32 = jnp.float32

    msg_mu, msg_sig = pl.pallas_call(
        _front_body,
        out_shape=(jax.ShapeDtypeStruct((NPTS, 64), f32),
                   jax.ShapeDtypeStruct((NPTS, 64), f32)),
    )(x, W1, r(b1), r(g1), r(be1),
      Wml, r(bml), r(gml), r(beml),
      Wsl, r(bsl), r(gsl), r(besl))

    nbr = _knn0(x)
    agg_mu = jnp.max(jnp.take(msg_mu, nbr, axis=0), axis=1)
    agg_sig = jnp.max(jnp.take(msg_sig, nbr, axis=0), axis=1)

    out, z_mu, z_sig = pl.pallas_call(
        _tail_body,
        out_shape=(jax.ShapeDtypeStruct((NB, 3 * P), f32),
                   jax.ShapeDtypeStruct((NB, 128), f32),
                   jax.ShapeDtypeStruct((NB, 128), f32)),
    )(agg_mu, agg_sig,
      Wmg, r(bmg), r(gmg), r(bemg),
      Wsg, r(bsg), r(gsg), r(besg),
      W3, r(b3), W4, r(b4))

    return (out.reshape(P * NB, 3), z_mu, z_sig, z_mu)


# speedup vs baseline: 3.2341x; 3.1856x over previous
import functools

import jax
import jax.numpy as jnp
from jax.experimental import pallas as pl
from jax.experimental.pallas import tpu as pltpu

EPS = 1e-5
SLOPE = 0.01
NB = 8
P = 2048
NPTS = NB * P
KNN = 16


def _bn(h, g, be):
    m = jnp.mean(h, axis=0, keepdims=True)
    v = jnp.mean((h - m) ** 2, axis=0, keepdims=True)
    return (h - m) * jax.lax.rsqrt(v + EPS) * g + be


def _mlp(h, W, b, g, be):
    h = jnp.dot(h, W, preferred_element_type=jnp.float32) + b
    h = jnp.where(h > 0.0, h, SLOPE * h)
    return _bn(h, g, be)


def _front_body(x_ref, W1_ref, b1_ref, g1_ref, be1_ref,
                Wml_ref, bml_ref, gml_ref, beml_ref,
                Wsl_ref, bsl_ref, gsl_ref, besl_ref,
                mu_ref, sig_ref):
    x = x_ref[...]
    h = _mlp(x, W1_ref[...], b1_ref[...], g1_ref[...], be1_ref[...])
    mu_ref[...] = _mlp(h, Wml_ref[...], bml_ref[...], gml_ref[...], beml_ref[...])
    sig_ref[...] = _mlp(h, Wsl_ref[...], bsl_ref[...], gsl_ref[...], besl_ref[...])


def _tail_body(aggm_ref, aggs_ref,
               Wmg_ref, bmg_ref, gmg_ref, bemg_ref,
               Wsg_ref, bsg_ref, gsg_ref, besg_ref,
               W3_ref, b3_ref, W4_ref, b4_ref,
               out_ref, zmu_ref, zsig_ref):
    zm = _mlp(aggm_ref[...], Wmg_ref[...], bmg_ref[...], gmg_ref[...], bemg_ref[...])
    zs = _mlp(aggs_ref[...], Wsg_ref[...], bsg_ref[...], gsg_ref[...], besg_ref[...])
    zm = jnp.max(zm.reshape(NB, P, 128), axis=1)
    zs = jnp.max(zs.reshape(NB, P, 128), axis=1)
    zmu_ref[...] = zm
    zsig_ref[...] = jnp.minimum(zs, 3.0)
    o = jnp.dot(zm, W3_ref[...], preferred_element_type=jnp.float32) + b3_ref[...]
    o = jnp.maximum(o, 0.0)
    out_ref[...] = jnp.dot(o, W4_ref[...], preferred_element_type=jnp.float32) + b4_ref[...]


KNN_ROWS = 512


def _knn_body(xr_ref, xt_ref, idx_ref):
    b = pl.program_id(0)
    rb = pl.program_id(1)
    rows = xr_ref[0]
    colsT = xt_ref[0]
    d2r = jnp.sum(rows * rows, axis=1, keepdims=True)
    d2c = jnp.sum(colsT * colsT, axis=0, keepdims=True)
    dist = d2r + d2c - 2.0 * jnp.dot(rows, colsT, preferred_element_type=jnp.float32)
    col = jax.lax.broadcasted_iota(jnp.int32, (KNN_ROWS, P), 1)
    row_g = rb * KNN_ROWS + jax.lax.broadcasted_iota(jnp.int32, (KNN_ROWS, 1), 0)
    dist = jnp.where(col == row_g, dist + 1e10, dist)
    picks = []
    for _ in range(KNN):
        m = jnp.min(dist, axis=1, keepdims=True)
        idx = jnp.min(jnp.where(dist == m, col, P), axis=1, keepdims=True)
        picks.append(idx)
        dist = jnp.where(col == idx, jnp.inf, dist)
    idx_ref[0] = jnp.concatenate(picks, axis=1) + b * P


def _knn(x):
    xg = x.reshape(NB, P, 3)
    xgT = xg.transpose(0, 2, 1)
    idx = pl.pallas_call(
        _knn_body,
        grid=(NB, P // KNN_ROWS),
        in_specs=[
            pl.BlockSpec((1, KNN_ROWS, 3), lambda b, r: (b, r, 0)),
            pl.BlockSpec((1, 3, P), lambda b, r: (b, 0, 0)),
        ],
        out_specs=pl.BlockSpec((1, KNN_ROWS, KNN), lambda b, r: (b, r, 0)),
        out_shape=jax.ShapeDtypeStruct((NB, P, KNN), jnp.int32),
    )(xg, xgT)
    return idx.reshape(NPTS, KNN)


def kernel(x, batch, category, W1, b1, g1, be1,
           Wml, bml, gml, beml, Wmg, bmg, gmg, bemg,
           Wsl, bsl, gsl, besl, Wsg, bsg, gsg, besg,
           W3, b3, W4, b4):
    r = lambda v: v.reshape(1, -1)
    f32 = jnp.float32

    msg_mu, msg_sig = pl.pallas_call(
        _front_body,
        out_shape=(jax.ShapeDtypeStruct((NPTS, 64), f32),
                   jax.ShapeDtypeStruct((NPTS, 64), f32)),
    )(x, W1, r(b1), r(g1), r(be1),
      Wml, r(bml), r(gml), r(beml),
      Wsl, r(bsl), r(gsl), r(besl))

    nbr = _knn(x)
    agg_mu = jnp.max(jnp.take(msg_mu, nbr, axis=0), axis=1)
    agg_sig = jnp.max(jnp.take(msg_sig, nbr, axis=0), axis=1)

    out, z_mu, z_sig = pl.pallas_call(
        _tail_body,
        out_shape=(jax.ShapeDtypeStruct((NB, 3 * P), f32),
                   jax.ShapeDtypeStruct((NB, 128), f32),
                   jax.ShapeDtypeStruct((NB, 128), f32)),
    )(agg_mu, agg_sig,
      Wmg, r(bmg), r(gmg), r(bemg),
      Wsg, r(bsg), r(gsg), r(besg),
      W3, r(b3), W4, r(b4))

    return (out.reshape(P * NB, 3), z_mu, z_sig, z_mu)


# speedup vs baseline: 9.7183x; 3.0049x over previous
import functools

import jax
import jax.numpy as jnp
from jax import lax
from jax.experimental import pallas as pl
from jax.experimental.pallas import tpu as pltpu
from jax.experimental.pallas import tpu_sc as plsc

EPS = 1e-5
SLOPE = 0.01
NB = 8
P = 2048
NPTS = NB * P
KNN = 16


def _bn(h, g, be):
    m = jnp.mean(h, axis=0, keepdims=True)
    v = jnp.mean((h - m) ** 2, axis=0, keepdims=True)
    return (h - m) * jax.lax.rsqrt(v + EPS) * g + be


def _mlp(h, W, b, g, be):
    h = jnp.dot(h, W, preferred_element_type=jnp.float32) + b
    h = jnp.where(h > 0.0, h, SLOPE * h)
    return _bn(h, g, be)


def _front_body(x_ref, W1_ref, b1_ref, g1_ref, be1_ref,
                Wml_ref, bml_ref, gml_ref, beml_ref,
                Wsl_ref, bsl_ref, gsl_ref, besl_ref,
                msg_ref):
    x = x_ref[...]
    h = _mlp(x, W1_ref[...], b1_ref[...], g1_ref[...], be1_ref[...])
    msg_ref[:, 0:64] = _mlp(h, Wml_ref[...], bml_ref[...], gml_ref[...], beml_ref[...])
    msg_ref[:, 64:128] = _mlp(h, Wsl_ref[...], bsl_ref[...], gsl_ref[...], besl_ref[...])


def _tail_body(agg_ref,
               Wmg_ref, bmg_ref, gmg_ref, bemg_ref,
               Wsg_ref, bsg_ref, gsg_ref, besg_ref,
               W3_ref, b3_ref, W4_ref, b4_ref,
               out_ref, zmu_ref, zsig_ref):
    zm = _mlp(agg_ref[:, 0:64], Wmg_ref[...], bmg_ref[...], gmg_ref[...], bemg_ref[...])
    zs = _mlp(agg_ref[:, 64:128], Wsg_ref[...], bsg_ref[...], gsg_ref[...], besg_ref[...])
    zm = jnp.max(zm.reshape(NB, P, 128), axis=1)
    zs = jnp.max(zs.reshape(NB, P, 128), axis=1)
    zmu_ref[...] = zm
    zsig_ref[...] = jnp.minimum(zs, 3.0)
    o = jnp.dot(zm, W3_ref[...], preferred_element_type=jnp.float32) + b3_ref[...]
    o = jnp.maximum(o, 0.0)
    out_ref[...] = jnp.dot(o, W4_ref[...], preferred_element_type=jnp.float32) + b4_ref[...]


KNN_ROWS = 512


def _knn_body(xr_ref, xt_ref, idx_ref):
    b = pl.program_id(0)
    rb = pl.program_id(1)
    rows = xr_ref[0]
    colsT = xt_ref[0]
    d2r = jnp.sum(rows * rows, axis=1, keepdims=True)
    d2c = jnp.sum(colsT * colsT, axis=0, keepdims=True)
    dist = d2r + d2c - 2.0 * jnp.dot(rows, colsT, preferred_element_type=jnp.float32)
    col = jax.lax.broadcasted_iota(jnp.int32, (KNN_ROWS, P), 1)
    row_g = rb * KNN_ROWS + jax.lax.broadcasted_iota(jnp.int32, (KNN_ROWS, 1), 0)
    dist = jnp.where(col == row_g, dist + 1e10, dist)
    picks = []
    for _ in range(KNN):
        m = jnp.min(dist, axis=1, keepdims=True)
        idx = jnp.min(jnp.where(dist == m, col, P), axis=1, keepdims=True)
        picks.append(idx)
        dist = jnp.where(col == idx, jnp.inf, dist)
    idx_ref[0] = jnp.concatenate(picks, axis=1) + b * P


def _knn(x):
    xg = x.reshape(NB, P, 3)
    xgT = xg.transpose(0, 2, 1)
    idx = pl.pallas_call(
        _knn_body,
        grid=(NB, P // KNN_ROWS),
        in_specs=[
            pl.BlockSpec((1, KNN_ROWS, 3), lambda b, r: (b, r, 0)),
            pl.BlockSpec((1, 3, P), lambda b, r: (b, 0, 0)),
        ],
        out_specs=pl.BlockSpec((1, KNN_ROWS, KNN), lambda b, r: (b, r, 0)),
        out_shape=jax.ShapeDtypeStruct((NB, P, KNN), jnp.int32),
    )(xg, xgT)
    return idx.reshape(NPTS, KNN)


NWORK = 32
NPW = NPTS // NWORK
GS = 4
NGRP = NPW // GS


def _scmax_body(msg_hbm, nbr_hbm, agg_hbm, idx_v, buf, out_v, sem):
    wid = lax.axis_index("s") * 2 + lax.axis_index("c")
    base = wid * NPW
    pltpu.sync_copy(nbr_hbm.at[pl.ds(wid * NGRP, NGRP)], idx_v)

    def group(g, _):
        pltpu.async_copy(msg_hbm.at[idx_v.at[g]], buf, sem).wait()
        for j in range(GS):
            for c in range(8):
                sl = pl.ds(c * 16, 16)
                am = buf[j * 16, sl]
                for r in range(1, 16):
                    am = jnp.maximum(am, buf[j * 16 + r, sl])
                out_v[g * GS + j, sl] = am
        return _

    lax.fori_loop(0, NGRP, group, None)
    pltpu.sync_copy(out_v, agg_hbm.at[pl.ds(base, NPW)])


def _sc_gather_max(msg, nbr):
    f32 = jnp.float32
    nbr_g = nbr.reshape(NWORK * NGRP, GS * KNN)
    run = pl.kernel(
        _scmax_body,
        out_type=jax.ShapeDtypeStruct((NPTS, 128), f32),
        mesh=plsc.VectorSubcoreMesh(core_axis_name="c", subcore_axis_name="s"),
        scratch_types=(
            pltpu.VMEM((NGRP, GS * KNN), jnp.int32),
            pltpu.VMEM((GS * KNN, 128), f32),
            pltpu.VMEM((NPW, 128), f32),
            pltpu.SemaphoreType.DMA,
        ),
    )
    return run(msg, nbr_g)


def kernel(x, batch, category, W1, b1, g1, be1,
           Wml, bml, gml, beml, Wmg, bmg, gmg, bemg,
           Wsl, bsl, gsl, besl, Wsg, bsg, gsg, besg,
           W3, b3, W4, b4):
    r = lambda v: v.reshape(1, -1)
    f32 = jnp.float32

    msg = pl.pallas_call(
        _front_body,
        out_shape=jax.ShapeDtypeStruct((NPTS, 128), f32),
    )(x, W1, r(b1), r(g1), r(be1),
      Wml, r(bml), r(gml), r(beml),
      Wsl, r(bsl), r(gsl), r(besl))

    nbr = _knn(x)
    agg = _sc_gather_max(msg, nbr)

    out, z_mu, z_sig = pl.pallas_call(
        _tail_body,
        out_shape=(jax.ShapeDtypeStruct((NB, 3 * P), f32),
                   jax.ShapeDtypeStruct((NB, 128), f32),
                   jax.ShapeDtypeStruct((NB, 128), f32)),
    )(agg,
      Wmg, r(bmg), r(gmg), r(bemg),
      Wsg, r(bsg), r(gsg), r(besg),
      W3, r(b3), W4, r(b4))

    return (out.reshape(P * NB, 3), z_mu, z_sig, z_mu)


# speedup vs baseline: 13.0030x; 1.3380x over previous
import functools

import jax
import jax.numpy as jnp
from jax import lax
from jax.experimental import pallas as pl
from jax.experimental.pallas import tpu as pltpu
from jax.experimental.pallas import tpu_sc as plsc

EPS = 1e-5
SLOPE = 0.01
NB = 8
P = 2048
NPTS = NB * P
KNN = 16


def _bn(h, g, be):
    m = jnp.mean(h, axis=0, keepdims=True)
    v = jnp.mean((h - m) ** 2, axis=0, keepdims=True)
    return (h - m) * jax.lax.rsqrt(v + EPS) * g + be


def _mlp(h, W, b, g, be):
    h = jnp.dot(h, W, preferred_element_type=jnp.float32) + b
    h = jnp.where(h > 0.0, h, SLOPE * h)
    return _bn(h, g, be)


def _front_body(x_ref, W1_ref, b1_ref, g1_ref, be1_ref,
                Wml_ref, bml_ref, gml_ref, beml_ref,
                Wsl_ref, bsl_ref, gsl_ref, besl_ref,
                msg_ref):
    x = x_ref[...]
    h = _mlp(x, W1_ref[...], b1_ref[...], g1_ref[...], be1_ref[...])
    msg_ref[:, 0:64] = _mlp(h, Wml_ref[...], bml_ref[...], gml_ref[...], beml_ref[...])
    msg_ref[:, 64:128] = _mlp(h, Wsl_ref[...], bsl_ref[...], gsl_ref[...], besl_ref[...])


def _tail_body(agg_ref,
               Wmg_ref, bmg_ref, gmg_ref, bemg_ref,
               Wsg_ref, bsg_ref, gsg_ref, besg_ref,
               W3_ref, b3_ref, W4_ref, b4_ref,
               out_ref, zmu_ref, zsig_ref):
    zm = _mlp(agg_ref[:, 0:64], Wmg_ref[...], bmg_ref[...], gmg_ref[...], bemg_ref[...])
    zs = _mlp(agg_ref[:, 64:128], Wsg_ref[...], bsg_ref[...], gsg_ref[...], besg_ref[...])
    zm = jnp.max(zm.reshape(NB, P, 128), axis=1)
    zs = jnp.max(zs.reshape(NB, P, 128), axis=1)
    zmu_ref[...] = zm
    zsig_ref[...] = jnp.minimum(zs, 3.0)
    o = jnp.dot(zm, W3_ref[...], preferred_element_type=jnp.float32) + b3_ref[...]
    o = jnp.maximum(o, 0.0)
    out_ref[...] = jnp.dot(o, W4_ref[...], preferred_element_type=jnp.float32) + b4_ref[...]


KNN_ROWS = 512


def _knn_body(xr_ref, xt_ref, idx_ref):
    b = pl.program_id(0)
    rb = pl.program_id(1)
    rows = xr_ref[0]
    colsT = xt_ref[0]
    d2r = jnp.sum(rows * rows, axis=1, keepdims=True)
    d2c = jnp.sum(colsT * colsT, axis=0, keepdims=True)
    dist = d2r + d2c - 2.0 * jnp.dot(rows, colsT, preferred_element_type=jnp.float32)
    col = jax.lax.broadcasted_iota(jnp.int32, (KNN_ROWS, P), 1)
    row_g = rb * KNN_ROWS + jax.lax.broadcasted_iota(jnp.int32, (KNN_ROWS, 1), 0)
    dist = jnp.where(col == row_g, dist + 1e10, dist)
    pb = jax.lax.bitcast_convert_type(jnp.maximum(dist, 0.0), jnp.int32)
    pb = (pb & jnp.int32(-2048)) | col
    big = jnp.int32(0x7FFFFFFF)
    t = jnp.full((KNN_ROWS, 1), -1, jnp.int32)
    picks = []
    for _ in range(KNN):
        t = jnp.min(jnp.where(pb > t, pb, big), axis=1, keepdims=True)
        picks.append(t & 2047)
    idx_ref[0] = jnp.concatenate(picks, axis=1) + b * P


def _knn(x):
    xg = x.reshape(NB, P, 3)
    xgT = xg.transpose(0, 2, 1)
    idx = pl.pallas_call(
        _knn_body,
        grid=(NB, P // KNN_ROWS),
        in_specs=[
            pl.BlockSpec((1, KNN_ROWS, 3), lambda b, r: (b, r, 0)),
            pl.BlockSpec((1, 3, P), lambda b, r: (b, 0, 0)),
        ],
        out_specs=pl.BlockSpec((1, KNN_ROWS, KNN), lambda b, r: (b, r, 0)),
        out_shape=jax.ShapeDtypeStruct((NB, P, KNN), jnp.int32),
    )(xg, xgT)
    return idx.reshape(NPTS, KNN)


NWORK = 32
NPW = NPTS // NWORK
GS = 8
NGRP = NPW // GS


def _scmax_body(msg_hbm, nbr_hbm, agg_hbm, idx_v, buf0, buf1, out_v, sem0, sem1):
    wid = lax.axis_index("s") * 2 + lax.axis_index("c")
    base = wid * NPW
    pltpu.sync_copy(nbr_hbm.at[pl.ds(wid * NGRP, NGRP)], idx_v)

    def fire(g, buf, sem):
        pltpu.async_copy(msg_hbm.at[idx_v.at[g]], buf, sem)

    def drain(buf, sem):
        pltpu.make_async_copy(msg_hbm.at[idx_v.at[0]], buf, sem).wait()

    def reduce_group(g, buf):
        for j in range(GS):
            for c in range(8):
                sl = pl.ds(c * 16, 16)
                am = buf[j * 16, sl]
                for r in range(1, 16):
                    am = jnp.maximum(am, buf[j * 16 + r, sl])
                out_v[g * GS + j, sl] = am

    fire(0, buf0, sem0)
    fire(1, buf1, sem1)

    def pair(i, _):
        g = 2 * i
        drain(buf0, sem0)
        reduce_group(g, buf0)
        @pl.when(g + 2 < NGRP)
        def _f0():
            fire(g + 2, buf0, sem0)
        drain(buf1, sem1)
        reduce_group(g + 1, buf1)
        @pl.when(g + 3 < NGRP)
        def _f1():
            fire(g + 3, buf1, sem1)
        return _

    lax.fori_loop(0, NGRP // 2, pair, None)
    pltpu.sync_copy(out_v, agg_hbm.at[pl.ds(base, NPW)])


def _sc_gather_max(msg, nbr):
    f32 = jnp.float32
    nbr_g = nbr.reshape(NWORK * NGRP, GS * KNN)
    run = pl.kernel(
        _scmax_body,
        out_type=jax.ShapeDtypeStruct((NPTS, 128), f32),
        mesh=plsc.VectorSubcoreMesh(core_axis_name="c", subcore_axis_name="s"),
        scratch_types=(
            pltpu.VMEM((NGRP, GS * KNN), jnp.int32),
            pltpu.VMEM((GS * KNN, 128), f32),
            pltpu.VMEM((GS * KNN, 128), f32),
            pltpu.VMEM((NPW, 128), f32),
            pltpu.SemaphoreType.DMA,
            pltpu.SemaphoreType.DMA,
        ),
    )
    return run(msg, nbr_g)


def kernel(x, batch, category, W1, b1, g1, be1,
           Wml, bml, gml, beml, Wmg, bmg, gmg, bemg,
           Wsl, bsl, gsl, besl, Wsg, bsg, gsg, besg,
           W3, b3, W4, b4):
    r = lambda v: v.reshape(1, -1)
    f32 = jnp.float32

    msg = pl.pallas_call(
        _front_body,
        out_shape=jax.ShapeDtypeStruct((NPTS, 128), f32),
    )(x, W1, r(b1), r(g1), r(be1),
      Wml, r(bml), r(gml), r(beml),
      Wsl, r(bsl), r(gsl), r(besl))

    nbr = _knn(x)
    agg = _sc_gather_max(msg, nbr)

    out, z_mu, z_sig = pl.pallas_call(
        _tail_body,
        out_shape=(jax.ShapeDtypeStruct((NB, 3 * P), f32),
                   jax.ShapeDtypeStruct((NB, 128), f32),
                   jax.ShapeDtypeStruct((NB, 128), f32)),
    )(agg,
      Wmg, r(bmg), r(gmg), r(bemg),
      Wsg, r(bsg), r(gsg), r(besg),
      W3, r(b3), W4, r(b4))

    return (out.reshape(P * NB, 3), z_mu, z_sig, z_mu)


# speedup vs baseline: 14.3668x; 1.1049x over previous
import functools

import jax
import jax.numpy as jnp
from jax import lax
from jax.experimental import pallas as pl
from jax.experimental.pallas import tpu as pltpu
from jax.experimental.pallas import tpu_sc as plsc

EPS = 1e-5
SLOPE = 0.01
NB = 8
P = 2048
NPTS = NB * P
KNN = 16


def _bn(h, g, be):
    m = jnp.mean(h, axis=0, keepdims=True)
    v = jnp.mean((h - m) ** 2, axis=0, keepdims=True)
    return (h - m) * jax.lax.rsqrt(v + EPS) * g + be


def _mlp(h, W, b, g, be):
    h = jnp.dot(h, W, preferred_element_type=jnp.float32) + b
    h = jnp.where(h > 0.0, h, SLOPE * h)
    return _bn(h, g, be)


def _front_body(x_ref, W1_ref, b1_ref, g1_ref, be1_ref,
                Wml_ref, bml_ref, gml_ref, beml_ref,
                Wsl_ref, bsl_ref, gsl_ref, besl_ref,
                msg_ref):
    x = x_ref[...]
    h = _mlp(x, W1_ref[...], b1_ref[...], g1_ref[...], be1_ref[...])
    msg_ref[:, 0:64] = _mlp(h, Wml_ref[...], bml_ref[...], gml_ref[...], beml_ref[...])
    msg_ref[:, 64:128] = _mlp(h, Wsl_ref[...], bsl_ref[...], gsl_ref[...], besl_ref[...])


def _tail_body(agg_a_ref, agg_b_ref,
               Wmg_ref, bmg_ref, gmg_ref, bemg_ref,
               Wsg_ref, bsg_ref, gsg_ref, besg_ref,
               W3_ref, b3_ref, W4_ref, b4_ref,
               out_ref, zmu_ref, zsig_ref):
    agg = jnp.concatenate([agg_a_ref[...], agg_b_ref[...]], axis=0)
    zm = _mlp(agg[:, 0:64], Wmg_ref[...], bmg_ref[...], gmg_ref[...], bemg_ref[...])
    zs = _mlp(agg[:, 64:128], Wsg_ref[...], bsg_ref[...], gsg_ref[...], besg_ref[...])
    zm = jnp.max(zm.reshape(NB, P, 128), axis=1)
    zs = jnp.max(zs.reshape(NB, P, 128), axis=1)
    zmu_ref[...] = zm
    zsig_ref[...] = jnp.minimum(zs, 3.0)
    o = jnp.dot(zm, W3_ref[...], preferred_element_type=jnp.float32) + b3_ref[...]
    o = jnp.maximum(o, 0.0)
    out_ref[...] = jnp.dot(o, W4_ref[...], preferred_element_type=jnp.float32) + b4_ref[...]


KNN_ROWS = 512
NBH = NB // 2
NPH = NBH * P


def _knn_body(cloud_off, xr_ref, xt_ref, idx_ref):
    b = pl.program_id(0) + cloud_off
    rb = pl.program_id(1)
    rows = xr_ref[0]
    colsT = xt_ref[0]
    d2r = jnp.sum(rows * rows, axis=1, keepdims=True)
    d2c = jnp.sum(colsT * colsT, axis=0, keepdims=True)
    dist = d2r + d2c - 2.0 * jnp.dot(rows, colsT, preferred_element_type=jnp.float32)
    col = jax.lax.broadcasted_iota(jnp.int32, (KNN_ROWS, P), 1)
    row_g = rb * KNN_ROWS + jax.lax.broadcasted_iota(jnp.int32, (KNN_ROWS, 1), 0)
    dist = jnp.where(col == row_g, dist + 1e10, dist)
    pb = jax.lax.bitcast_convert_type(jnp.maximum(dist, 0.0), jnp.int32)
    pb = (pb & jnp.int32(-2048)) | col
    big = jnp.int32(0x7FFFFFFF)
    t = jnp.full((KNN_ROWS, 1), -1, jnp.int32)
    picks = []
    for _ in range(KNN):
        t = jnp.min(jnp.where(pb > t, pb, big), axis=1, keepdims=True)
        picks.append(t & 2047)
    idx_ref[0] = jnp.concatenate(picks, axis=1) + b * P


def _knn_half(xg_h, xgT_h, cloud_off):
    idx = pl.pallas_call(
        functools.partial(_knn_body, cloud_off),
        grid=(NBH, P // KNN_ROWS),
        in_specs=[
            pl.BlockSpec((1, KNN_ROWS, 3), lambda b, r: (b, r, 0)),
            pl.BlockSpec((1, 3, P), lambda b, r: (b, 0, 0)),
        ],
        out_specs=pl.BlockSpec((1, KNN_ROWS, KNN), lambda b, r: (b, r, 0)),
        out_shape=jax.ShapeDtypeStruct((NBH, P, KNN), jnp.int32),
    )(xg_h, xgT_h)
    return idx.reshape(NPH, KNN)


NWORK = 32
NPW = NPH // NWORK
GS = 8
NGRP = NPW // GS


def _scmax_body(msg_hbm, nbr_hbm, agg_hbm, idx_v, buf0, buf1, out_v, sem0, sem1):
    wid = lax.axis_index("s") * 2 + lax.axis_index("c")
    base = wid * NPW
    pltpu.sync_copy(nbr_hbm.at[pl.ds(wid * NGRP, NGRP)], idx_v)

    def fire(g, buf, sem):
        pltpu.async_copy(msg_hbm.at[idx_v.at[g]], buf, sem)

    def drain(buf, sem):
        pltpu.make_async_copy(msg_hbm.at[idx_v.at[0]], buf, sem).wait()

    def reduce_group(g, buf):
        for j in range(GS):
            for c in range(8):
                sl = pl.ds(c * 16, 16)
                am = buf[j * 16, sl]
                for r in range(1, 16):
                    am = jnp.maximum(am, buf[j * 16 + r, sl])
                out_v[g * GS + j, sl] = am

    fire(0, buf0, sem0)
    fire(1, buf1, sem1)

    def pair(i, _):
        g = 2 * i
        drain(buf0, sem0)
        reduce_group(g, buf0)
        @pl.when(g + 2 < NGRP)
        def _f0():
            fire(g + 2, buf0, sem0)
        drain(buf1, sem1)
        reduce_group(g + 1, buf1)
        @pl.when(g + 3 < NGRP)
        def _f1():
            fire(g + 3, buf1, sem1)
        return _

    lax.fori_loop(0, NGRP // 2, pair, None)
    pltpu.sync_copy(out_v, agg_hbm.at[pl.ds(base, NPW)])


def _sc_gather_max(msg, nbr_h):
    f32 = jnp.float32
    nbr_g = nbr_h.reshape(NWORK * NGRP, GS * KNN)
    run = pl.kernel(
        _scmax_body,
        out_type=jax.ShapeDtypeStruct((NPH, 128), f32),
        mesh=plsc.VectorSubcoreMesh(core_axis_name="c", subcore_axis_name="s"),
        scratch_types=(
            pltpu.VMEM((NGRP, GS * KNN), jnp.int32),
            pltpu.VMEM((GS * KNN, 128), f32),
            pltpu.VMEM((GS * KNN, 128), f32),
            pltpu.VMEM((NPW, 128), f32),
            pltpu.SemaphoreType.DMA,
            pltpu.SemaphoreType.DMA,
        ),
    )
    return run(msg, nbr_g)


def kernel(x, batch, category, W1, b1, g1, be1,
           Wml, bml, gml, beml, Wmg, bmg, gmg, bemg,
           Wsl, bsl, gsl, besl, Wsg, bsg, gsg, besg,
           W3, b3, W4, b4):
    r = lambda v: v.reshape(1, -1)
    f32 = jnp.float32

    msg = pl.pallas_call(
        _front_body,
        out_shape=jax.ShapeDtypeStruct((NPTS, 128), f32),
    )(x, W1, r(b1), r(g1), r(be1),
      Wml, r(bml), r(gml), r(beml),
      Wsl, r(bsl), r(gsl), r(besl))

    xg = x.reshape(NB, P, 3)
    xgT = xg.transpose(0, 2, 1)
    nbr_a = _knn_half(xg[:NBH], xgT[:NBH], 0)
    agg_a = _sc_gather_max(msg, nbr_a)
    nbr_b = _knn_half(xg[NBH:], xgT[NBH:], NBH)
    agg_b = _sc_gather_max(msg, nbr_b)

    out, z_mu, z_sig = pl.pallas_call(
        _tail_body,
        out_shape=(jax.ShapeDtypeStruct((NB, 3 * P), f32),
                   jax.ShapeDtypeStruct((NB, 128), f32),
                   jax.ShapeDtypeStruct((NB, 128), f32)),
    )(agg_a, agg_b,
      Wmg, r(bmg), r(gmg), r(bemg),
      Wsg, r(bsg), r(gsg), r(besg),
      W3, r(b3), W4, r(b4))

    return (out.reshape(P * NB, 3), z_mu, z_sig, z_mu)
